# Initial kernel scaffold; baseline (speedup 1.0000x reference)
import jax
import jax.numpy as jnp
from jax.experimental import pallas as pl


def kernel(x, batch):
    raise NotImplementedError("write your pallas kernel here")


# speedup vs baseline: 4.7111x; 4.7111x over previous
import functools

import jax
import jax.numpy as jnp
from jax import lax
from jax.experimental import pallas as pl
from jax.experimental.pallas import tpu as pltpu
from jax.experimental.pallas import tpu_sc as plsc

N = 100000
D = 128
S = 256
CHUNK = 80
NCHUNKS = N // CHUNK
NC = 2
NS = 16
NW = NC * NS
KMAX = (NCHUNKS + NW - 1) // NW


def _sc_pool(x, batch):
    mesh = plsc.VectorSubcoreMesh(core_axis_name="c", subcore_axis_name="s",
                                  num_cores=NC, num_subcores=NS)

    @functools.partial(
        pl.kernel,
        out_type=(
            jax.ShapeDtypeStruct((NC, S, D), jnp.float32),
            jax.ShapeDtypeStruct((NC, S), jnp.float32),
        ),
        mesh=mesh,
        scratch_types=[
            pltpu.VMEM((CHUNK, D), jnp.float32),
            pltpu.VMEM((CHUNK,), jnp.int32),
            pltpu.VMEM((CHUNK,), jnp.float32),
            pltpu.VMEM((16,), jnp.float32),
            pltpu.VMEM_SHARED((S, D), jnp.float32),
            pltpu.VMEM_SHARED((S,), jnp.float32),
        ],
    )
    def pool(x_hbm, batch_hbm, sums_hbm, cnts_hbm,
             xbuf, idxbuf, ones, zb, acc_sh, cnt_sh):
        cid = lax.axis_index("c")
        sid = lax.axis_index("s")
        wid = sid * NC + cid

        zero16 = jnp.zeros((16,), jnp.float32)
        one16 = jnp.full((16,), 1.0, jnp.float32)

        zb[...] = zero16

        def init_row(i, _):
            @pl.when(i < CHUNK // 16)
            def _():
                ones[pl.ds(i * 16, 16)] = one16

            @pl.when(i < 16)
            def _():
                for j in range(D // 16):
                    xbuf[i, pl.ds(j * 16, 16)] = zero16
            return 0

        lax.fori_loop(0, 16, init_row, 0)

        pltpu.sync_copy(xbuf.at[pl.ds(0, 16)], acc_sh.at[pl.ds(sid * 16, 16)])
        pltpu.sync_copy(zb, cnt_sh.at[pl.ds(sid * 16, 16)])
        plsc.subcore_barrier()

        def chunk_body(k, _):
            c = k * NW + wid

            @pl.when(c < NCHUNKS)
            def _():
                base = c * CHUNK
                pltpu.sync_copy(x_hbm.at[pl.ds(base, CHUNK)], xbuf)
                pltpu.sync_copy(batch_hbm.at[pl.ds(base, CHUNK)], idxbuf)
                pltpu.sync_copy(xbuf, acc_sh.at[idxbuf], add=True)
                pltpu.sync_copy(ones, cnt_sh.at[idxbuf], add=True)
            return 0

        lax.fori_loop(0, KMAX, chunk_body, 0)
        plsc.subcore_barrier()

        pltpu.sync_copy(acc_sh.at[pl.ds(sid * 16, 16)],
                        sums_hbm.at[cid, pl.ds(sid * 16, 16)])

        @pl.when(sid == 0)
        def _():
            pltpu.sync_copy(cnt_sh, cnts_hbm.at[cid])

    return pool(x, batch)


def _combine_body(s_ref, c_ref, o_ref):
    s = s_ref[0] + s_ref[1]
    c = c_ref[0] + c_ref[1]
    o_ref[...] = s / jnp.maximum(c, 1.0)


def kernel(x, batch):
    sums, cnts = _sc_pool(x, batch.astype(jnp.int32))
    out = pl.pallas_call(
        _combine_body,
        out_shape=jax.ShapeDtypeStruct((S, D), jnp.float32),
    )(sums, cnts[:, :, None])
    return out


# speedup vs baseline: 8.1742x; 1.7351x over previous
import functools

import jax
import jax.numpy as jnp
from jax import lax
from jax.experimental import pallas as pl
from jax.experimental.pallas import tpu as pltpu
from jax.experimental.pallas import tpu_sc as plsc

N = 100000
D = 128
S = 256
CHUNK = 200
HALF = CHUNK // 2
NCHUNKS = N // CHUNK
NC = 2
NS = 16
NW = NC * NS
KMAX = (NCHUNKS + NW - 1) // NW
FULL_K = NCHUNKS // NW
REM = NCHUNKS - FULL_K * NW
NBUF = 2


def _sc_pool(x, batch2d):
    mesh = plsc.VectorSubcoreMesh(core_axis_name="c", subcore_axis_name="s",
                                  num_cores=NC, num_subcores=NS)

    @functools.partial(
        pl.kernel,
        out_type=(
            jax.ShapeDtypeStruct((NC, S, D), jnp.float32),
            jax.ShapeDtypeStruct((NC, S), jnp.float32),
        ),
        mesh=mesh,
        scratch_types=[
            pltpu.VMEM((NBUF, CHUNK, D), jnp.float32),
            pltpu.VMEM((2 * KMAX, HALF), jnp.int32),
            pltpu.VMEM((7 * 16,), jnp.float32),
            pltpu.VMEM((16,), jnp.float32),
            pltpu.VMEM_SHARED((S, D), jnp.float32),
            pltpu.VMEM_SHARED((S,), jnp.float32),
            pltpu.SemaphoreType.DMA,
            pltpu.SemaphoreType.DMA,
        ],
    )
    def pool(x_hbm, b_hbm, sums_hbm, cnts_hbm,
             xbuf, idxs, ones, zb, acc_sh, cnt_sh, sem0, sem1):
        cid = lax.axis_index("c")
        sid = lax.axis_index("s")
        wid = sid * NC + cid
        sems = [sem0, sem1]

        zero16 = jnp.zeros((16,), jnp.float32)
        one16 = jnp.full((16,), 1.0, jnp.float32)

        zb[...] = zero16
        for i in range(7):
            ones[pl.ds(i * 16, 16)] = one16

        def zrow(i, _):
            for j in range(D // 16):
                xbuf[0, i, pl.ds(j * 16, 16)] = zero16
            return 0

        lax.fori_loop(0, 16, zrow, 0)

        pltpu.sync_copy(xbuf.at[0, pl.ds(0, 16)],
                        acc_sh.at[pl.ds(sid * 16, 16)])

        @pl.when(sid == 0)
        def _():
            def zc(i, _):
                pltpu.sync_copy(zb, cnt_sh.at[pl.ds(i * 16, 16)])
                return 0
            lax.fori_loop(0, S // 16, zc, 0)

        plsc.subcore_barrier()

        has_extra = wid < REM

        def issue(k):
            b = k % NBUF
            c = k * NW + wid
            base = c * CHUNK
            dx = pltpu.async_copy(x_hbm.at[pl.ds(base, CHUNK)],
                                  xbuf.at[b], sems[b])
            di = pltpu.async_copy(b_hbm.at[pl.ds(c * 2, 2)],
                                  idxs.at[pl.ds(2 * k, 2)], sems[b])
            return dx, di

        def consume(k, descs):
            b = k % NBUF
            dx, di = descs
            dx.wait()
            di.wait()
            for h in range(2):
                idx_row = idxs.at[2 * k + h]
                pltpu.sync_copy(xbuf.at[b, pl.ds(h * HALF, HALF)],
                                acc_sh.at[idx_row], add=True)
                pltpu.sync_copy(ones.at[pl.ds(0, HALF)],
                                cnt_sh.at[idx_row], add=True)

        pending = {}
        for k in range(min(NBUF, FULL_K)):
            pending[k] = issue(k)
        for k in range(KMAX):
            if k < FULL_K:
                consume(k, pending.pop(k))
                nxt = k + NBUF
                if nxt < FULL_K:
                    pending[nxt] = issue(nxt)
                elif nxt < KMAX:
                    @pl.when(has_extra)
                    def _(nxt=nxt):
                        dx, di = issue(nxt)
                        consume(nxt, (dx, di))

        plsc.subcore_barrier()

        pltpu.sync_copy(acc_sh.at[pl.ds(sid * 16, 16)],
                        sums_hbm.at[cid, pl.ds(sid * 16, 16)])

        @pl.when(sid == 0)
        def _():
            pltpu.sync_copy(cnt_sh, cnts_hbm.at[cid])

    return pool(x, batch2d)


def _combine_body(s_ref, c_ref, o_ref):
    s = s_ref[0] + s_ref[1]
    c = c_ref[0] + c_ref[1]
    o_ref[...] = s / jnp.maximum(c, 1.0)


def kernel(x, batch):
    batch2d = batch.astype(jnp.int32).reshape(N // HALF, HALF)
    sums, cnts = _sc_pool(x, batch2d)
    out = pl.pallas_call(
        _combine_body,
        out_shape=jax.ShapeDtypeStruct((S, D), jnp.float32),
    )(sums, cnts[:, :, None])
    return out
